# baseline (device time: 73416 ns/iter reference)
import jax
import jax.numpy as jnp
from jax import lax
from jax.experimental import pallas as pl
from jax.experimental.pallas import tpu as pltpu

N_DEV = 4
B = 2
SQ = 512
SKV = 512
H_LOC = 8
DH = 64
D_LOC = H_LOC * DH
D_MODEL = 768
BLK = 64


def kernel(x, Wq, K_ext, V_ext, Wo):
    def body(x_ref, wq_ref, k_ref, v_ref, wo_ref, out_ref,
             ctx_ref, send_sems, recv_sems):
        my = lax.axis_index("i")
        left = lax.rem(my + N_DEV - 1, N_DEV)
        right = lax.rem(my + 1, N_DEV)

        barrier = pltpu.get_barrier_semaphore()
        for nbr in (left, right):
            pl.semaphore_signal(
                barrier, inc=1,
                device_id=(nbr,), device_id_type=pl.DeviceIdType.MESH,
            )
        pl.semaphore_wait(barrier, 2)

        qb = lax.broadcasted_iota(jnp.int32, (SQ, SKV), 0) // BLK
        kb = lax.broadcasted_iota(jnp.int32, (SQ, SKV), 1) // BLK
        mask = kb <= qb

        wq_loc = wq_ref[:, pl.ds(my * D_LOC, D_LOC)].astype(jnp.bfloat16)
        for b in range(B):
            xb = x_ref[b].astype(jnp.bfloat16)
            q_b = jnp.dot(xb, wq_loc,
                          preferred_element_type=jnp.float32)
            for h in range(H_LOC):
                qh = q_b[:, h * DH:(h + 1) * DH].astype(jnp.bfloat16)
                kh = k_ref[b, :, h, :].astype(jnp.bfloat16)
                vh = v_ref[b, :, h, :].astype(jnp.bfloat16)
                s = lax.dot_general(
                    qh, kh, (((1,), (1,)), ((), ())),
                    preferred_element_type=jnp.float32,
                ) * 0.125
                s = jnp.where(mask, s, -1e9)
                m = jnp.max(s, axis=-1, keepdims=True)
                w = jnp.exp(s - m)
                w = w / jnp.sum(w, axis=-1, keepdims=True)
                ctx = jnp.dot(w.astype(jnp.bfloat16), vh,
                              preferred_element_type=jnp.float32)
                ctx_ref[0, b, :, h * DH:(h + 1) * DH] = ctx.astype(jnp.bfloat16)

        for h in range(N_DEV - 1):
            rdma = pltpu.make_async_remote_copy(
                src_ref=ctx_ref.at[h],
                dst_ref=ctx_ref.at[h + 1],
                send_sem=send_sems.at[h],
                recv_sem=recv_sems.at[h],
                device_id=(right,),
                device_id_type=pl.DeviceIdType.MESH,
            )
            rdma.start()
            rdma.wait()

        for b in range(B):
            acc = jnp.zeros((SQ, D_MODEL), jnp.float32)
            for s in range(N_DEV):
                origin = lax.rem(my + N_DEV - s, N_DEV)
                wo_slab = wo_ref[pl.ds(origin * D_LOC, D_LOC), :].astype(
                    jnp.bfloat16)
                acc = acc + jnp.dot(ctx_ref[s, b], wo_slab,
                                    preferred_element_type=jnp.float32)
            out_ref[b] = acc

    return pl.pallas_call(
        body,
        out_shape=jax.ShapeDtypeStruct((B, SQ, D_MODEL), jnp.float32),
        in_specs=[pl.BlockSpec(memory_space=pltpu.VMEM)] * 5,
        out_specs=pl.BlockSpec(memory_space=pltpu.VMEM),
        scratch_shapes=[
            pltpu.VMEM((N_DEV, B, SQ, D_LOC), jnp.bfloat16),
            pltpu.SemaphoreType.DMA((N_DEV - 1,)),
            pltpu.SemaphoreType.DMA((N_DEV - 1,)),
        ],
        compiler_params=pltpu.CompilerParams(collective_id=0),
    )(x, Wq, K_ext, V_ext, Wo)


# device time: 49948 ns/iter; 1.4698x vs baseline; 1.4698x over previous
import jax
import jax.numpy as jnp
from jax import lax
from jax.experimental import pallas as pl
from jax.experimental.pallas import tpu as pltpu

N_DEV = 4
B = 2
SQ = 512
SKV = 512
H_LOC = 8
DH = 64
D_LOC = H_LOC * DH
D_MODEL = 768
BLK = 64
ROWS = B * SQ


def kernel(x, Wq, K_ext, V_ext, Wo):
    def body(x_ref, wq_ref, k_ref, v_ref, wo_ref, out_ref,
             p_ref, stage1, stage2, send_sems, recv_sems):
        my = lax.axis_index("i")
        xp = 3 - my
        yp = my ^ 1
        xc = my // 2
        yc = xc ^ (my % 2)

        barrier = pltpu.get_barrier_semaphore()
        for nbr in (xp, yp):
            pl.semaphore_signal(
                barrier, inc=1,
                device_id=(nbr,), device_id_type=pl.DeviceIdType.MESH,
            )
        pl.semaphore_wait(barrier, 2)

        qb = lax.broadcasted_iota(jnp.int32, (SQ, SKV), 0) // BLK
        kb = lax.broadcasted_iota(jnp.int32, (SQ, SKV), 1) // BLK
        mask = kb <= qb

        wq_loc = wq_ref[:, pl.ds(my * D_LOC, D_LOC)].astype(jnp.bfloat16)
        wo_loc = wo_ref[pl.ds(my * D_LOC, D_LOC), :].astype(jnp.bfloat16)

        def partial_for_batch(b):
            xb = x_ref[b].astype(jnp.bfloat16)
            q_b = jnp.dot(xb, wq_loc,
                          preferred_element_type=jnp.float32)
            heads = []
            for h in range(H_LOC):
                qh = q_b[:, h * DH:(h + 1) * DH].astype(jnp.bfloat16)
                kh = k_ref[b, :, h, :].astype(jnp.bfloat16)
                vh = v_ref[b, :, h, :].astype(jnp.bfloat16)
                s = lax.dot_general(
                    qh, kh, (((1,), (1,)), ((), ())),
                    preferred_element_type=jnp.float32,
                ) * 0.125
                s = jnp.where(mask, s, -1e9)
                m = jnp.max(s, axis=-1, keepdims=True)
                w = jnp.exp(s - m)
                w = w / jnp.sum(w, axis=-1, keepdims=True)
                heads.append(jnp.dot(w.astype(jnp.bfloat16), vh,
                                     preferred_element_type=jnp.float32))
            ctx = jnp.concatenate(heads, axis=1).astype(jnp.bfloat16)
            return jnp.dot(ctx, wo_loc,
                           preferred_element_type=jnp.float32)

        keep1A = xc * 256
        send1A = (1 - xc) * 256
        keep2A = keep1A + yc * 128
        send2A = keep1A + (1 - yc) * 128
        keep1B = SQ + yc * 256
        send1B = SQ + (1 - yc) * 256
        keep2B = keep1B + xc * 128
        send2B = keep1B + (1 - xc) * 128

        def exch(idx, src_start, n, peer, dst_ref, dst_start=None):
            if dst_start is None:
                dst = dst_ref
            else:
                dst = dst_ref.at[pl.ds(dst_start, n)]
            return pltpu.make_async_remote_copy(
                src_ref=p_ref.at[pl.ds(src_start, n)],
                dst_ref=dst,
                send_sem=send_sems.at[idx],
                recv_sem=recv_sems.at[idx],
                device_id=(peer,),
                device_id_type=pl.DeviceIdType.MESH,
            )

        def add_into(start, n, stage):
            blk = p_ref[pl.ds(start, n), :].astype(jnp.float32)
            blk = blk + stage[...].astype(jnp.float32)
            p_ref[pl.ds(start, n), :] = blk.astype(jnp.bfloat16)

        p_ref[pl.ds(0, SQ), :] = partial_for_batch(0).astype(jnp.bfloat16)
        r1a = exch(0, send1A, 256, xp, stage1.at[0])
        r1a.start()

        p_ref[pl.ds(SQ, SQ), :] = partial_for_batch(1).astype(jnp.bfloat16)
        r1b = exch(1, send1B, 256, yp, stage1.at[1])
        r1b.start()

        r1a.wait()
        add_into(keep1A, 256, stage1.at[0])
        r2a = exch(2, send2A, 128, yp, stage2.at[0])
        r2a.start()

        r1b.wait()
        add_into(keep1B, 256, stage1.at[1])
        r2b = exch(3, send2B, 128, xp, stage2.at[1])
        r2b.start()

        r2a.wait()
        add_into(keep2A, 128, stage2.at[0])
        r3a = exch(4, keep2A, 128, yp, p_ref, keep2A)
        r3a.start()

        r2b.wait()
        add_into(keep2B, 128, stage2.at[1])
        r3b = exch(5, keep2B, 128, xp, p_ref, keep2B)
        r3b.start()

        r3a.wait()
        r4a = exch(6, keep1A, 256, xp, p_ref, keep1A)
        r4a.start()

        r3b.wait()
        r4b = exch(7, keep1B, 256, yp, p_ref, keep1B)
        r4b.start()

        r4a.wait()
        r4b.wait()

        out_ref[0] = p_ref[pl.ds(0, SQ), :].astype(jnp.float32)
        out_ref[1] = p_ref[pl.ds(SQ, SQ), :].astype(jnp.float32)

    return pl.pallas_call(
        body,
        out_shape=jax.ShapeDtypeStruct((B, SQ, D_MODEL), jnp.float32),
        in_specs=[pl.BlockSpec(memory_space=pltpu.VMEM)] * 5,
        out_specs=pl.BlockSpec(memory_space=pltpu.VMEM),
        scratch_shapes=[
            pltpu.VMEM((ROWS, D_MODEL), jnp.bfloat16),
            pltpu.VMEM((2, 256, D_MODEL), jnp.bfloat16),
            pltpu.VMEM((2, 128, D_MODEL), jnp.bfloat16),
            pltpu.SemaphoreType.DMA((8,)),
            pltpu.SemaphoreType.DMA((8,)),
        ],
        compiler_params=pltpu.CompilerParams(collective_id=0),
    )(x, Wq, K_ext, V_ext, Wo)
